# baseline (speedup 1.0000x reference)
import jax
import jax.numpy as jnp
from jax.experimental import pallas as pl
from jax.experimental.pallas import tpu as pltpu

N_TOKENS = 4096
INPUT_SIZE = 8192
NUM_COLS = 2048
K_ACTIVE = 40

ROWS_BLK = 1024
J_BLK = 1024
NB = N_TOKENS // ROWS_BLK
J_STEPS = INPUT_SIZE // J_BLK
SEARCH_ITERS_PER_STEP = -(-32 // J_STEPS)
MAX_FINITE_BITS = 0x7F7FFFFF


def _body(x_ref, w_ref, o_ref, acc_ref, lo_ref, hi_ref):
    i = pl.program_id(0)
    j = pl.program_id(1)
    par = jax.lax.rem(i, 2)

    @pl.when(i < NB)
    def _matmul():
        part = jax.lax.dot_general(
            x_ref[...], w_ref[...],
            dimension_numbers=(((1,), (1,)), ((), ())),
            preferred_element_type=jnp.float32,
        )

        @pl.when(j == 0)
        def _set():
            acc_ref[par] = part

        @pl.when(j > 0)
        def _add():
            acc_ref[par] += part

    @pl.when(i > 0)
    def _search():
        acc = acc_ref[1 - par]

        @pl.when(j == 0)
        def _init():
            lo_ref[...] = jnp.zeros_like(lo_ref)
            hi_ref[...] = jnp.full_like(hi_ref, MAX_FINITE_BITS)

        lo = lo_ref[...]
        hi = hi_ref[...]
        for _ in range(SEARCH_ITERS_PER_STEP):
            mid = lo + (hi - lo + 1) // 2
            t = jax.lax.bitcast_convert_type(mid, jnp.float32)
            cnt = jnp.sum((acc >= t).astype(jnp.int32), axis=1, keepdims=True)
            ge = cnt >= K_ACTIVE
            lo = jnp.where(ge, mid, lo)
            hi = jnp.where(ge, hi, mid - 1)
        lo_ref[...] = lo
        hi_ref[...] = hi

        @pl.when(j == J_STEPS - 1)
        def _emit():
            thr = jax.lax.bitcast_convert_type(lo, jnp.float32)
            o_ref[...] = (acc >= thr).astype(jnp.int8)


def _pallas_kwta(input_vector, connections):
    return pl.pallas_call(
        _body,
        grid=(NB + 1, J_STEPS),
        in_specs=[
            pl.BlockSpec((ROWS_BLK, J_BLK),
                         lambda i, j: (jnp.minimum(i, NB - 1), j)),
            pl.BlockSpec((NUM_COLS, J_BLK), lambda i, j: (0, j)),
        ],
        out_specs=pl.BlockSpec((ROWS_BLK, NUM_COLS),
                               lambda i, j: (jnp.maximum(i - 1, 0), 0)),
        out_shape=jax.ShapeDtypeStruct((N_TOKENS, NUM_COLS), jnp.int8),
        scratch_shapes=[
            pltpu.VMEM((2, ROWS_BLK, NUM_COLS), jnp.float32),
            pltpu.VMEM((ROWS_BLK, 1), jnp.int32),
            pltpu.VMEM((ROWS_BLK, 1), jnp.int32),
        ],
    )(input_vector, connections)


def kernel(input_vector, connections):
    return _pallas_kwta(input_vector, connections).astype(jnp.bool_)


# speedup vs baseline: 1.0212x; 1.0212x over previous
import jax
import jax.numpy as jnp
from jax.experimental import pallas as pl
from jax.experimental.pallas import tpu as pltpu

N_TOKENS = 4096
INPUT_SIZE = 8192
NUM_COLS = 2048
K_ACTIVE = 40

ROWS_BLK = 1024
J_BLK = 1024
NB = N_TOKENS // ROWS_BLK
J_STEPS = INPUT_SIZE // J_BLK
SEARCH_ITERS_PER_STEP = -(-32 // J_STEPS)
MAX_FINITE_BITS = 0x7F7FFFFF


def _body(x_ref, w_ref, o_ref, acc0_ref, acc1_ref, lo_ref, hi_ref):
    i = pl.program_id(0)
    j = pl.program_id(1)
    par = jax.lax.rem(i, 2)

    def stage(accw_ref, accr_ref):
        part = jax.lax.dot_general(
            x_ref[...], w_ref[...],
            dimension_numbers=(((1,), (1,)), ((), ())),
            preferred_element_type=jnp.float32,
        )
        accw_ref[...] = jnp.where(j == 0, part, accw_ref[...] + part)

        acc = accr_ref[...]
        lo = jnp.where(j == 0, jnp.zeros_like(lo_ref), lo_ref[...])
        hi = jnp.where(j == 0,
                       jnp.full_like(hi_ref, MAX_FINITE_BITS), hi_ref[...])
        for _ in range(SEARCH_ITERS_PER_STEP):
            mid = lo + (hi - lo + 1) // 2
            t = jax.lax.bitcast_convert_type(mid, jnp.float32)
            cnt = jnp.sum((acc >= t).astype(jnp.int32), axis=1, keepdims=True)
            ge = cnt >= K_ACTIVE
            lo = jnp.where(ge, mid, lo)
            hi = jnp.where(ge, hi, mid - 1)
        lo_ref[...] = lo
        hi_ref[...] = hi

        @pl.when(j == J_STEPS - 1)
        def _emit():
            thr = jax.lax.bitcast_convert_type(lo, jnp.float32)
            o_ref[...] = (acc >= thr).astype(jnp.int8)

    @pl.when(par == 0)
    def _even():
        stage(acc0_ref, acc1_ref)

    @pl.when(par == 1)
    def _odd():
        stage(acc1_ref, acc0_ref)


def _pallas_kwta(input_vector, connections):
    return pl.pallas_call(
        _body,
        grid=(NB + 1, J_STEPS),
        in_specs=[
            pl.BlockSpec((ROWS_BLK, J_BLK),
                         lambda i, j: (jnp.minimum(i, NB - 1), j)),
            pl.BlockSpec((NUM_COLS, J_BLK), lambda i, j: (0, j)),
        ],
        out_specs=pl.BlockSpec((ROWS_BLK, NUM_COLS),
                               lambda i, j: (jnp.maximum(i - 1, 0), 0)),
        out_shape=jax.ShapeDtypeStruct((N_TOKENS, NUM_COLS), jnp.int8),
        scratch_shapes=[
            pltpu.VMEM((ROWS_BLK, NUM_COLS), jnp.float32),
            pltpu.VMEM((ROWS_BLK, NUM_COLS), jnp.float32),
            pltpu.VMEM((ROWS_BLK, 1), jnp.int32),
            pltpu.VMEM((ROWS_BLK, 1), jnp.int32),
        ],
    )(input_vector, connections)


def kernel(input_vector, connections):
    return _pallas_kwta(input_vector, connections).astype(jnp.bool_)


# speedup vs baseline: 1.2555x; 1.2295x over previous
import jax
import jax.numpy as jnp
from jax.experimental import pallas as pl
from jax.experimental.pallas import tpu as pltpu

N_TOKENS = 4096
INPUT_SIZE = 8192
NUM_COLS = 2048
K_ACTIVE = 40

ROWS_BLK = 1024
J_BLK = 1024
NB = N_TOKENS // ROWS_BLK
J_STEPS = INPUT_SIZE // J_BLK


def _body(x_ref, w_ref, o_ref, acc_ref):
    j = pl.program_id(1)

    @pl.when(j == 0)
    def _zero():
        acc_ref[...] = jnp.zeros_like(acc_ref)

    acc_ref[...] += jax.lax.dot_general(
        x_ref[...], w_ref[...],
        dimension_numbers=(((1,), (1,)), ((), ())),
        preferred_element_type=jnp.float32,
    )

    @pl.when(j == J_STEPS - 1)
    def _finish():
        acc = acc_ref[...]
        lo = jax.lax.bitcast_convert_type(
            jnp.min(acc, axis=1, keepdims=True), jnp.int32)
        hi = jax.lax.bitcast_convert_type(
            jnp.max(acc, axis=1, keepdims=True), jnp.int32)

        def cond(lohi):
            lo, hi = lohi
            return jnp.any(lo < hi)

        def step(lohi):
            lo, hi = lohi
            mid = lo + (hi - lo + 1) // 2
            t = jax.lax.bitcast_convert_type(mid, jnp.float32)
            cnt = jnp.sum((acc >= t).astype(jnp.int32), axis=1, keepdims=True)
            ge = cnt >= K_ACTIVE
            lo = jnp.where(ge, mid, lo)
            hi = jnp.where(ge, hi, mid - 1)
            return lo, hi

        lo, _ = jax.lax.while_loop(cond, step, (lo, hi))
        thr = jax.lax.bitcast_convert_type(lo, jnp.float32)
        o_ref[...] = (acc >= thr).astype(jnp.int8)


def _pallas_kwta(input_vector, connections):
    return pl.pallas_call(
        _body,
        grid=(NB, J_STEPS),
        in_specs=[
            pl.BlockSpec((ROWS_BLK, J_BLK), lambda i, j: (i, j)),
            pl.BlockSpec((NUM_COLS, J_BLK), lambda i, j: (0, j)),
        ],
        out_specs=pl.BlockSpec((ROWS_BLK, NUM_COLS), lambda i, j: (i, 0)),
        out_shape=jax.ShapeDtypeStruct((N_TOKENS, NUM_COLS), jnp.int8),
        scratch_shapes=[pltpu.VMEM((ROWS_BLK, NUM_COLS), jnp.float32)],
    )(input_vector, connections)


def kernel(input_vector, connections):
    return _pallas_kwta(input_vector, connections).astype(jnp.bool_)
